# baseline (speedup 1.0000x reference)
import functools

import jax
import jax.numpy as jnp
from jax import lax
from jax.experimental import pallas as pl
from jax.experimental.pallas import tpu as pltpu
from jax.experimental.pallas import tpu_sc as plsc

_K = 8
_NPAD = 102400
_CBLK = 4096
_QBLK = 256
_NGRP = _NPAD // 32
_NSEL = 16
_NCAND = _NSEL * 32
_BIG = 3.0e38
_SENT = 1.0e4
_NQ = 1024
_NW = 32
_QPW = _NQ // _NW


def _select_body(x_ref, c_ref, c2_ref, gids_ref, gmin_ref):
    j = pl.program_id(1)
    xb = x_ref[...].astype(jnp.bfloat16)
    cb = c_ref[...].astype(jnp.bfloat16)
    m = jax.lax.dot_general(xb, cb, (((1,), (1,)), ((), ())),
                            preferred_element_type=jnp.float32)
    s = c2_ref[...] - 2.0 * m
    gm = s[:, 0:128]
    for a in range(1, _CBLK // 128):
        gm = jnp.minimum(gm, s[:, a * 128:(a + 1) * 128])
    gmin_ref[:, pl.ds(j * 128, 128)] = gm

    @pl.when(j == pl.num_programs(1) - 1)
    def _finalize():
        iota = jax.lax.broadcasted_iota(jnp.int32, (_QBLK, _NGRP), 1)
        for it in range(_NSEL):
            sg = gmin_ref[...]
            v = jnp.min(sg, axis=1)
            idx = jnp.min(jnp.where(sg == v[:, None], iota, jnp.int32(2**30)),
                          axis=1)
            gids_ref[:, it:it + 1] = idx[:, None]
            gmin_ref[...] = jnp.where(iota == idx[:, None], _BIG, sg)


def _candidate_groups(x, ctrs_pad, c2_pad):
    n = x.shape[0]
    grid = (n // _QBLK, _NPAD // _CBLK)
    return pl.pallas_call(
        _select_body,
        grid=grid,
        in_specs=[
            pl.BlockSpec((_QBLK, 32), lambda i, j: (i, 0)),
            pl.BlockSpec((_CBLK, 32), lambda i, j: (j, 0)),
            pl.BlockSpec((1, _CBLK), lambda i, j: (0, j)),
        ],
        out_specs=pl.BlockSpec((_QBLK, _NSEL), lambda i, j: (i, 0)),
        out_shape=jax.ShapeDtypeStruct((n, _NSEL), jnp.int32),
        scratch_shapes=[pltpu.VMEM((_QBLK, _NGRP), jnp.float32)],
        compiler_params=pltpu.CompilerParams(
            dimension_semantics=("arbitrary", "arbitrary")),
    )(x, ctrs_pad, c2_pad)


def _gather_body(cand_hbm, ctrs_hbm, rowsg_hbm, idx_v, rows_v, sem):
    c = lax.axis_index("c")
    s = lax.axis_index("s")
    wid = s * 2 + c

    def qbody(qi, carry):
        q = wid * _QPW + qi
        pltpu.sync_copy(cand_hbm.at[q], idx_v)
        copies = [pltpu.async_copy(ctrs_hbm.at[idx_v.at[i]],
                                   rows_v.at[pl.ds(i * 128, 128)], sem)
                  for i in range(4)]
        for cp in copies:
            cp.wait()
        pltpu.sync_copy(rows_v, rowsg_hbm.at[q])
        return carry

    lax.fori_loop(0, _QPW, qbody, 0)


def _gather_candidates(cand3, ctrs_pad):
    kfn = functools.partial(
        pl.kernel,
        mesh=plsc.VectorSubcoreMesh(core_axis_name="c", subcore_axis_name="s"),
        out_type=jax.ShapeDtypeStruct((_NQ, _NCAND, 32), jnp.float32),
        scratch_types=[
            pltpu.VMEM((4, 128), jnp.int32),
            pltpu.VMEM((_NCAND, 32), jnp.float32),
            pltpu.SemaphoreType.DMA,
        ],
        compiler_params=pltpu.CompilerParams(use_tc_tiling_on_sc=False),
    )(_gather_body)
    return kfn(cand3, ctrs_pad)


def _rescore_body(xb_ref, cc_ref, cand_ref, idx8_ref):
    xb = xb_ref[...]
    cc = cc_ref[...]
    ccb = cc.astype(jnp.bfloat16).astype(jnp.float32)
    m = jnp.sum(ccb * xb[:, None, :], axis=2)
    c2 = jnp.sum(cc * cc, axis=2)
    sc = c2 - 2.0 * m
    cnd = cand_ref[...]
    for k in range(_K):
        v = jnp.min(sc, axis=1)
        hit = sc == v[:, None]
        cid = jnp.min(jnp.where(hit, cnd, jnp.int32(2**30)), axis=1)
        idx8_ref[:, k:k + 1] = cid[:, None]
        sc = jnp.where(hit & (cnd == cid[:, None]), _BIG, sc)


def _rescore_top8(xb32, rows_g, cand):
    qb2 = 32
    return pl.pallas_call(
        _rescore_body,
        grid=(_NQ // qb2,),
        in_specs=[
            pl.BlockSpec((qb2, 32), lambda i: (i, 0)),
            pl.BlockSpec((qb2, _NCAND, 32), lambda i: (i, 0, 0)),
            pl.BlockSpec((qb2, _NCAND), lambda i: (i, 0)),
        ],
        out_specs=pl.BlockSpec((qb2, _K), lambda i: (i, 0)),
        out_shape=jax.ShapeDtypeStruct((_NQ, _K), jnp.int32),
        compiler_params=pltpu.CompilerParams(
            dimension_semantics=("arbitrary",)),
    )(xb32, rows_g, cand)


def _combine_body(idx8_hbm, x_hbm, ctrs_hbm, wts2_hbm, off_hbm, out_hbm,
                  id8_v, w_v, off_v, c8_v, xrow_v, outrow_v, sem):
    c = lax.axis_index("c")
    s = lax.axis_index("s")
    wid = s * 2 + c

    def qbody(qi, carry):
        q = wid * _QPW + qi
        pltpu.sync_copy(idx8_hbm.at[q], id8_v)
        pltpu.sync_copy(x_hbm.at[q], xrow_v)
        cw = pltpu.async_copy(wts2_hbm.at[id8_v], w_v, sem)
        co = pltpu.async_copy(off_hbm.at[id8_v], off_v, sem)
        cc8 = pltpu.async_copy(ctrs_hbm.at[id8_v], c8_v, sem)
        cw.wait()
        co.wait()
        cc8.wait()

        xr0 = xrow_v[pl.ds(0, 16)]
        xr1 = xrow_v[pl.ds(16, 16)]
        acc0 = jnp.zeros((16,), jnp.float32)
        acc1 = jnp.zeros((16,), jnp.float32)
        for k in range(_K):
            xc0 = xr0 - c8_v[k, pl.ds(0, 16)]
            xc1 = xr1 - c8_v[k, pl.ds(16, 16)]
            for d in range(32):
                sd = xc0[d] if d < 16 else xc1[d - 16]
                acc0 = acc0 + sd * w_v[k, pl.ds(d * 32, 16)]
                acc1 = acc1 + sd * w_v[k, pl.ds(d * 32 + 16, 16)]
            acc0 = acc0 + off_v[k, pl.ds(0, 16)]
            acc1 = acc1 + off_v[k, pl.ds(16, 16)]
        outrow_v[pl.ds(0, 16)] = acc0
        outrow_v[pl.ds(16, 16)] = acc1
        pltpu.sync_copy(outrow_v, out_hbm.at[q])
        return carry

    lax.fori_loop(0, _QPW, qbody, 0)


def _gather_combine(idx8, x, ctrs, wts2, offsets):
    kfn = functools.partial(
        pl.kernel,
        mesh=plsc.VectorSubcoreMesh(core_axis_name="c", subcore_axis_name="s"),
        out_type=jax.ShapeDtypeStruct((_NQ, 32), jnp.float32),
        scratch_types=[
            pltpu.VMEM((_K,), jnp.int32),
            pltpu.VMEM((_K, 1024), jnp.float32),
            pltpu.VMEM((_K, 32), jnp.float32),
            pltpu.VMEM((_K, 32), jnp.float32),
            pltpu.VMEM((32,), jnp.float32),
            pltpu.VMEM((32,), jnp.float32),
            pltpu.SemaphoreType.DMA,
        ],
        compiler_params=pltpu.CompilerParams(use_tc_tiling_on_sc=False),
    )(_combine_body)
    return kfn(idx8, x, ctrs, wts2, offsets)


def kernel(x, ctrs, wts, offsets):
    n = x.shape[0]
    nf = ctrs.shape[0]
    ctrs_pad = jnp.pad(ctrs, ((0, _NPAD - nf), (0, 0)),
                       constant_values=_SENT)
    c2_pad = jnp.sum(ctrs_pad * ctrs_pad, axis=1)[None, :]

    gids = _candidate_groups(x, ctrs_pad, c2_pad)

    a = jnp.arange(_CBLK // 128, dtype=jnp.int32) * 128
    cand = ((gids // 128) * _CBLK + gids % 128)[:, :, None] + a[None, None, :]
    cand = cand.reshape(n, _NCAND)
    cand3 = cand.reshape(n, 4, 128)

    rows_g = _gather_candidates(cand3, ctrs_pad)

    xb32 = x.astype(jnp.bfloat16).astype(jnp.float32)
    idx8 = _rescore_top8(xb32, rows_g, cand)

    wts2 = wts.reshape(nf, 1024)
    return _gather_combine(idx8, x, ctrs, wts2, offsets)
